# speedup vs baseline: 49.5402x; 49.5402x over previous
import jax
import jax.numpy as jnp
from jax.experimental import pallas as pl

_TH = 16


def _body(v_ref, o_ref):
    v = v_ref[0]
    d = v.shape[0]
    m1 = jnp.min(v, axis=0)
    iota = jax.lax.broadcasted_iota(jnp.int32, v.shape, 0)
    idx = jnp.min(jnp.where(v == m1[None], iota, jnp.int32(d)), axis=0)
    m2 = jnp.min(jnp.where(iota == idx[None], jnp.float32(jnp.inf), v), axis=0)
    o_ref[0, 0] = m2 - m1


def kernel(volume):
    b, d, h, w = volume.shape
    grid = (b, h // _TH)
    return pl.pallas_call(
        _body,
        grid=grid,
        in_specs=[pl.BlockSpec((1, d, _TH, w), lambda i, j: (i, 0, j, 0))],
        out_specs=pl.BlockSpec((1, 1, _TH, w), lambda i, j: (i, 0, j, 0)),
        out_shape=jax.ShapeDtypeStruct((b, 1, h, w), volume.dtype),
    )(volume)


# speedup vs baseline: 63.3587x; 1.2789x over previous
import jax
import jax.numpy as jnp
from jax.experimental import pallas as pl

_TH = 16


def _body(v_ref, o_ref):
    d = v_ref.shape[1]
    a = v_ref[0, 0]
    b = v_ref[0, 1]
    m1 = jnp.minimum(a, b)
    m2 = jnp.maximum(a, b)
    for i in range(2, d):
        x = v_ref[0, i]
        m2 = jnp.minimum(m2, jnp.maximum(m1, x))
        m1 = jnp.minimum(m1, x)
    o_ref[0, 0] = m2 - m1


def kernel(volume):
    b, d, h, w = volume.shape
    grid = (b, h // _TH)
    return pl.pallas_call(
        _body,
        grid=grid,
        in_specs=[pl.BlockSpec((1, d, _TH, w), lambda i, j: (i, 0, j, 0))],
        out_specs=pl.BlockSpec((1, 1, _TH, w), lambda i, j: (i, 0, j, 0)),
        out_shape=jax.ShapeDtypeStruct((b, 1, h, w), volume.dtype),
    )(volume)


# speedup vs baseline: 72.2048x; 1.1396x over previous
import jax
import jax.numpy as jnp
from jax.experimental import pallas as pl

_TH = 32


def _body(v_ref, o_ref):
    d = v_ref.shape[1]
    a = v_ref[0, 0]
    b = v_ref[0, 1]
    m1 = jnp.minimum(a, b)
    m2 = jnp.maximum(a, b)
    for i in range(2, d):
        x = v_ref[0, i]
        m2 = jnp.minimum(m2, jnp.maximum(m1, x))
        m1 = jnp.minimum(m1, x)
    o_ref[0, 0] = m2 - m1


def kernel(volume):
    b, d, h, w = volume.shape
    grid = (b, h // _TH)
    return pl.pallas_call(
        _body,
        grid=grid,
        in_specs=[pl.BlockSpec((1, d, _TH, w), lambda i, j: (i, 0, j, 0))],
        out_specs=pl.BlockSpec((1, 1, _TH, w), lambda i, j: (i, 0, j, 0)),
        out_shape=jax.ShapeDtypeStruct((b, 1, h, w), volume.dtype),
    )(volume)
